# baseline (speedup 1.0000x reference)
import functools

import jax
import jax.numpy as jnp
from jax import lax
from jax.experimental import pallas as pl
from jax.experimental.pallas import tpu as pltpu
from jax.experimental.pallas import tpu_sc as plsc

BATCH = 16384
D_MODEL = 512
LANES = 16

NUM_CORES = 2
NUM_SUBCORES = 16
NUM_WORKERS = NUM_CORES * NUM_SUBCORES
B_PER_W = BATCH // NUM_WORKERS
CHUNK = 64
NCHUNKS = B_PER_W // CHUNK


def _body(x_hbm, idx_hbm, tbl_hbm, out_hbm, idx_v, xbuf, rbuf, gsem):
    wid = lax.axis_index("s") * NUM_CORES + lax.axis_index("c")
    base = wid * B_PER_W
    pltpu.sync_copy(idx_hbm.at[pl.ds(base, B_PER_W)], idx_v)

    for c in range(NCHUNKS):
        row0 = base + c * CHUNK
        gather = pltpu.async_copy(
            tbl_hbm.at[idx_v.at[pl.ds(c * CHUNK, CHUNK)]], rbuf, gsem)
        pltpu.sync_copy(x_hbm.at[pl.ds(row0, CHUNK)], xbuf)
        gather.wait()

        def add_row(i, _):
            for j in range(D_MODEL // LANES):
                sl = pl.ds(j * LANES, LANES)
                rbuf[i, sl] = rbuf[i, sl] + xbuf[i, sl]
            return 0

        lax.fori_loop(0, CHUNK, add_row, 0)
        pltpu.sync_copy(rbuf, out_hbm.at[pl.ds(row0, CHUNK)])


def kernel(x, context_info, context_emb_weight):
    mesh = plsc.VectorSubcoreMesh(core_axis_name="c", subcore_axis_name="s")
    kfn = functools.partial(
        pl.kernel,
        mesh=mesh,
        out_type=jax.ShapeDtypeStruct((BATCH, D_MODEL), jnp.float32),
        scratch_types=[
            pltpu.VMEM((B_PER_W,), jnp.int32),
            pltpu.VMEM((CHUNK, D_MODEL), jnp.float32),
            pltpu.VMEM((CHUNK, D_MODEL), jnp.float32),
            pltpu.SemaphoreType.DMA,
        ],
    )(_body)
    return kfn(x, context_info.astype(jnp.int32), context_emb_weight)


# speedup vs baseline: 1.1416x; 1.1416x over previous
import functools

import jax
import jax.numpy as jnp
from jax import lax
from jax.experimental import pallas as pl
from jax.experimental.pallas import tpu as pltpu
from jax.experimental.pallas import tpu_sc as plsc

BATCH = 16384
D_MODEL = 512
LANES = 16

NUM_CORES = 2
NUM_SUBCORES = 16
NUM_WORKERS = NUM_CORES * NUM_SUBCORES
B_PER_W = BATCH // NUM_WORKERS
CHUNK = 32
NCHUNKS = B_PER_W // CHUNK
NBUF = 2


def _body(x_hbm, idx_hbm, tbl_hbm, out_hbm,
          idx_v, xbuf, rbuf, gsem, xsem, ssem):
    wid = lax.axis_index("s") * NUM_CORES + lax.axis_index("c")
    base = wid * B_PER_W
    pltpu.sync_copy(idx_hbm.at[pl.ds(base, B_PER_W)], idx_v)

    gathers = {}
    xloads = {}
    stores = {}
    for c in range(NCHUNKS + 1):
        if c < NCHUNKS:
            b = c % NBUF
            if c - NBUF in stores:
                stores.pop(c - NBUF).wait()
            row0 = base + c * CHUNK
            gathers[c] = pltpu.async_copy(
                tbl_hbm.at[idx_v.at[pl.ds(c * CHUNK, CHUNK)]],
                rbuf.at[b], gsem.at[b])
            xloads[c] = pltpu.async_copy(
                x_hbm.at[pl.ds(row0, CHUNK)], xbuf.at[b], xsem.at[b])
        if c >= 1:
            cc = c - 1
            b = cc % NBUF
            gathers.pop(cc).wait()
            xloads.pop(cc).wait()

            def add_row(i, _):
                for j in range(D_MODEL // LANES):
                    sl = pl.ds(j * LANES, LANES)
                    rbuf[b, i, sl] = rbuf[b, i, sl] + xbuf[b, i, sl]
                return 0

            lax.fori_loop(0, CHUNK, add_row, 0)
            stores[cc] = pltpu.async_copy(
                rbuf.at[b], out_hbm.at[pl.ds(base + cc * CHUNK, CHUNK)],
                ssem.at[b])
    for c in sorted(stores):
        stores.pop(c).wait()


def kernel(x, context_info, context_emb_weight):
    mesh = plsc.VectorSubcoreMesh(core_axis_name="c", subcore_axis_name="s")
    kfn = functools.partial(
        pl.kernel,
        mesh=mesh,
        out_type=jax.ShapeDtypeStruct((BATCH, D_MODEL), jnp.float32),
        scratch_types=[
            pltpu.VMEM((B_PER_W,), jnp.int32),
            pltpu.VMEM((NBUF, CHUNK, D_MODEL), jnp.float32),
            pltpu.VMEM((NBUF, CHUNK, D_MODEL), jnp.float32),
            pltpu.SemaphoreType.DMA((NBUF,)),
            pltpu.SemaphoreType.DMA((NBUF,)),
            pltpu.SemaphoreType.DMA((NBUF,)),
        ],
    )(_body)
    return kfn(x, context_info.astype(jnp.int32), context_emb_weight)


# speedup vs baseline: 1.1921x; 1.0443x over previous
import functools

import jax
import jax.numpy as jnp
from jax import lax
from jax.experimental import pallas as pl
from jax.experimental.pallas import tpu as pltpu
from jax.experimental.pallas import tpu_sc as plsc

BATCH = 16384
D_MODEL = 512
LANES = 16

NUM_CORES = 2
NUM_SUBCORES = 16
NUM_WORKERS = NUM_CORES * NUM_SUBCORES
B_PER_W = BATCH // NUM_WORKERS
CHUNK = 32
NCHUNKS = B_PER_W // CHUNK
NBUF = 3


def _body(x_hbm, idx_hbm, tbl_hbm, out_hbm,
          idx_v, xbuf, rbuf, gsem, xsem, ssem):
    wid = lax.axis_index("s") * NUM_CORES + lax.axis_index("c")
    base = wid * B_PER_W
    pltpu.sync_copy(idx_hbm.at[pl.ds(base, B_PER_W)], idx_v)

    gathers = {}
    xloads = {}
    stores = {}
    for c in range(NCHUNKS + 1):
        if c < NCHUNKS:
            b = c % NBUF
            if c - NBUF in stores:
                stores.pop(c - NBUF).wait()
            row0 = base + c * CHUNK
            gathers[c] = pltpu.async_copy(
                tbl_hbm.at[idx_v.at[pl.ds(c * CHUNK, CHUNK)]],
                rbuf.at[b], gsem.at[b])
            xloads[c] = pltpu.async_copy(
                x_hbm.at[pl.ds(row0, CHUNK)], xbuf.at[b], xsem.at[b])
        if c >= 1:
            cc = c - 1
            b = cc % NBUF
            gathers.pop(cc).wait()
            xloads.pop(cc).wait()

            def add_row(i, _):
                for j in range(D_MODEL // LANES):
                    sl = pl.ds(j * LANES, LANES)
                    rbuf[b, i, sl] = rbuf[b, i, sl] + xbuf[b, i, sl]
                return 0

            lax.fori_loop(0, CHUNK, add_row, 0)
            stores[cc] = pltpu.async_copy(
                rbuf.at[b], out_hbm.at[pl.ds(base + cc * CHUNK, CHUNK)],
                ssem.at[b])
    for c in sorted(stores):
        stores.pop(c).wait()


def kernel(x, context_info, context_emb_weight):
    mesh = plsc.VectorSubcoreMesh(core_axis_name="c", subcore_axis_name="s")
    kfn = functools.partial(
        pl.kernel,
        mesh=mesh,
        out_type=jax.ShapeDtypeStruct((BATCH, D_MODEL), jnp.float32),
        scratch_types=[
            pltpu.VMEM((B_PER_W,), jnp.int32),
            pltpu.VMEM((NBUF, CHUNK, D_MODEL), jnp.float32),
            pltpu.VMEM((NBUF, CHUNK, D_MODEL), jnp.float32),
            pltpu.SemaphoreType.DMA((NBUF,)),
            pltpu.SemaphoreType.DMA((NBUF,)),
            pltpu.SemaphoreType.DMA((NBUF,)),
        ],
    )(_body)
    return kfn(x, context_info.astype(jnp.int32), context_emb_weight)


# speedup vs baseline: 1.2298x; 1.0316x over previous
import functools

import jax
import jax.numpy as jnp
from jax import lax
from jax.experimental import pallas as pl
from jax.experimental.pallas import tpu as pltpu
from jax.experimental.pallas import tpu_sc as plsc

BATCH = 16384
D_MODEL = 512
LANES = 16

NUM_CORES = 2
NUM_SUBCORES = 16
NUM_WORKERS = NUM_CORES * NUM_SUBCORES
B_PER_W = BATCH // NUM_WORKERS
CHUNK = 32
NCHUNKS = B_PER_W // CHUNK
NBUF = 3


def _body(x_hbm, idx_hbm, tbl_hbm, out_hbm,
          idx_v, xbuf, rbuf, gsem, xsem, ssem):
    wid = lax.axis_index("s") * NUM_CORES + lax.axis_index("c")
    base = wid * B_PER_W
    pltpu.sync_copy(idx_hbm.at[pl.ds(base, B_PER_W)], idx_v)

    gathers = {}
    xloads = {}
    stores = {}
    for c in range(NCHUNKS + 1):
        if c < NCHUNKS:
            b = c % NBUF
            if c - NBUF in stores:
                stores.pop(c - NBUF).wait()
            row0 = base + c * CHUNK
            gathers[c] = pltpu.async_copy(
                tbl_hbm.at[idx_v.at[pl.ds(c * CHUNK, CHUNK)]],
                rbuf.at[b], gsem.at[b])
            xloads[c] = pltpu.async_copy(
                x_hbm.at[pl.ds(row0, CHUNK)], xbuf.at[b], xsem.at[b])
        if c >= 1:
            cc = c - 1
            b = cc % NBUF
            gathers.pop(cc).wait()
            xloads.pop(cc).wait()

            @plsc.parallel_loop(0, CHUNK, step=1)
            def add_row(i):
                for j in range(D_MODEL // LANES):
                    sl = pl.ds(j * LANES, LANES)
                    rbuf[b, i, sl] = rbuf[b, i, sl] + xbuf[b, i, sl]
            stores[cc] = pltpu.async_copy(
                rbuf.at[b], out_hbm.at[pl.ds(base + cc * CHUNK, CHUNK)],
                ssem.at[b])
    for c in sorted(stores):
        stores.pop(c).wait()


def kernel(x, context_info, context_emb_weight):
    mesh = plsc.VectorSubcoreMesh(core_axis_name="c", subcore_axis_name="s")
    kfn = functools.partial(
        pl.kernel,
        mesh=mesh,
        out_type=jax.ShapeDtypeStruct((BATCH, D_MODEL), jnp.float32),
        scratch_types=[
            pltpu.VMEM((B_PER_W,), jnp.int32),
            pltpu.VMEM((NBUF, CHUNK, D_MODEL), jnp.float32),
            pltpu.VMEM((NBUF, CHUNK, D_MODEL), jnp.float32),
            pltpu.SemaphoreType.DMA((NBUF,)),
            pltpu.SemaphoreType.DMA((NBUF,)),
            pltpu.SemaphoreType.DMA((NBUF,)),
        ],
    )(_body)
    return kfn(x, context_info.astype(jnp.int32), context_emb_weight)


# speedup vs baseline: 1.2301x; 1.0002x over previous
import functools

import jax
import jax.numpy as jnp
from jax import lax
from jax.experimental import pallas as pl
from jax.experimental.pallas import tpu as pltpu
from jax.experimental.pallas import tpu_sc as plsc

BATCH = 16384
D_MODEL = 512
LANES = 16

NUM_CORES = 2
NUM_SUBCORES = 16
NUM_WORKERS = NUM_CORES * NUM_SUBCORES
B_PER_W = BATCH // NUM_WORKERS
CHUNK = 32
NCHUNKS = B_PER_W // CHUNK
NBUF = 2
NOBUF = 2


def _body(x_hbm, idx_hbm, tbl_hbm, out_hbm,
          idx_v, xbuf, rbuf, obuf, gsem, xsem, ssem):
    wid = lax.axis_index("s") * NUM_CORES + lax.axis_index("c")
    base = wid * B_PER_W
    pltpu.sync_copy(idx_hbm.at[pl.ds(base, B_PER_W)], idx_v)

    def issue_loads(c):
        b = c % NBUF
        g = pltpu.async_copy(
            tbl_hbm.at[idx_v.at[pl.ds(c * CHUNK, CHUNK)]],
            rbuf.at[b], gsem.at[b])
        xl = pltpu.async_copy(
            x_hbm.at[pl.ds(base + c * CHUNK, CHUNK)], xbuf.at[b],
            xsem.at[b])
        return g, xl

    gathers = {}
    xloads = {}
    stores = {}
    for c in range(NBUF):
        gathers[c], xloads[c] = issue_loads(c)

    for c in range(NCHUNKS):
        b = c % NBUF
        o = c % NOBUF
        gathers.pop(c).wait()
        xloads.pop(c).wait()
        if c - NOBUF in stores:
            stores.pop(c - NOBUF).wait()

        @plsc.parallel_loop(0, CHUNK, step=1)
        def add_row(i):
            for j in range(D_MODEL // LANES):
                sl = pl.ds(j * LANES, LANES)
                obuf[o, i, sl] = rbuf[b, i, sl] + xbuf[b, i, sl]

        stores[c] = pltpu.async_copy(
            obuf.at[o], out_hbm.at[pl.ds(base + c * CHUNK, CHUNK)],
            ssem.at[o])
        if c + NBUF < NCHUNKS:
            gathers[c + NBUF], xloads[c + NBUF] = issue_loads(c + NBUF)
    for c in sorted(stores):
        stores.pop(c).wait()


def kernel(x, context_info, context_emb_weight):
    mesh = plsc.VectorSubcoreMesh(core_axis_name="c", subcore_axis_name="s")
    kfn = functools.partial(
        pl.kernel,
        mesh=mesh,
        out_type=jax.ShapeDtypeStruct((BATCH, D_MODEL), jnp.float32),
        scratch_types=[
            pltpu.VMEM((B_PER_W,), jnp.int32),
            pltpu.VMEM((NBUF, CHUNK, D_MODEL), jnp.float32),
            pltpu.VMEM((NBUF, CHUNK, D_MODEL), jnp.float32),
            pltpu.VMEM((NOBUF, CHUNK, D_MODEL), jnp.float32),
            pltpu.SemaphoreType.DMA((NBUF,)),
            pltpu.SemaphoreType.DMA((NBUF,)),
            pltpu.SemaphoreType.DMA((NOBUF,)),
        ],
    )(_body)
    return kfn(x, context_info.astype(jnp.int32), context_emb_weight)


# speedup vs baseline: 1.2394x; 1.0076x over previous
import functools

import jax
import jax.numpy as jnp
from jax import lax
from jax.experimental import pallas as pl
from jax.experimental.pallas import tpu as pltpu
from jax.experimental.pallas import tpu_sc as plsc

BATCH = 16384
D_MODEL = 512
LANES = 16

NUM_CORES = 2
NUM_SUBCORES = 16
NUM_WORKERS = NUM_CORES * NUM_SUBCORES
B_PER_W = BATCH // NUM_WORKERS
CHUNK = 32
NCHUNKS = B_PER_W // CHUNK
NGBUF = 3
NXBUF = 2
NOBUF = 2


def _body(x_hbm, idx_hbm, tbl_hbm, out_hbm,
          idx_v, xbuf, rbuf, obuf, gsem, xsem, ssem):
    wid = lax.axis_index("s") * NUM_CORES + lax.axis_index("c")
    base = wid * B_PER_W
    pltpu.sync_copy(idx_hbm.at[pl.ds(base, B_PER_W)], idx_v)

    def issue_gather(c):
        return pltpu.async_copy(
            tbl_hbm.at[idx_v.at[pl.ds(c * CHUNK, CHUNK)]],
            rbuf.at[c % NGBUF], gsem.at[c % NGBUF])

    def issue_xload(c):
        return pltpu.async_copy(
            x_hbm.at[pl.ds(base + c * CHUNK, CHUNK)],
            xbuf.at[c % NXBUF], xsem.at[c % NXBUF])

    gathers = {}
    xloads = {}
    stores = {}
    for c in range(NGBUF):
        gathers[c] = issue_gather(c)
    for c in range(NXBUF):
        xloads[c] = issue_xload(c)

    for c in range(NCHUNKS):
        bg = c % NGBUF
        bx = c % NXBUF
        o = c % NOBUF
        gathers.pop(c).wait()
        xloads.pop(c).wait()
        if c - NOBUF in stores:
            stores.pop(c - NOBUF).wait()

        @plsc.parallel_loop(0, CHUNK, step=1)
        def add_row(i):
            for j in range(D_MODEL // LANES):
                sl = pl.ds(j * LANES, LANES)
                obuf[o, i, sl] = rbuf[bg, i, sl] + xbuf[bx, i, sl]

        stores[c] = pltpu.async_copy(
            obuf.at[o], out_hbm.at[pl.ds(base + c * CHUNK, CHUNK)],
            ssem.at[o])
        if c + NGBUF < NCHUNKS:
            gathers[c + NGBUF] = issue_gather(c + NGBUF)
        if c + NXBUF < NCHUNKS:
            xloads[c + NXBUF] = issue_xload(c + NXBUF)
    for c in sorted(stores):
        stores.pop(c).wait()


def kernel(x, context_info, context_emb_weight):
    mesh = plsc.VectorSubcoreMesh(core_axis_name="c", subcore_axis_name="s")
    kfn = functools.partial(
        pl.kernel,
        mesh=mesh,
        out_type=jax.ShapeDtypeStruct((BATCH, D_MODEL), jnp.float32),
        scratch_types=[
            pltpu.VMEM((B_PER_W,), jnp.int32),
            pltpu.VMEM((NXBUF, CHUNK, D_MODEL), jnp.float32),
            pltpu.VMEM((NGBUF, CHUNK, D_MODEL), jnp.float32),
            pltpu.VMEM((NOBUF, CHUNK, D_MODEL), jnp.float32),
            pltpu.SemaphoreType.DMA((NGBUF,)),
            pltpu.SemaphoreType.DMA((NXBUF,)),
            pltpu.SemaphoreType.DMA((NOBUF,)),
        ],
    )(_body)
    return kfn(x, context_info.astype(jnp.int32), context_emb_weight)


# speedup vs baseline: 1.2832x; 1.0353x over previous
import functools

import jax
import jax.numpy as jnp
from jax import lax
from jax.experimental import pallas as pl
from jax.experimental.pallas import tpu as pltpu
from jax.experimental.pallas import tpu_sc as plsc

BATCH = 16384
D_MODEL = 512
LANES = 16

NUM_CORES = 2
NUM_SUBCORES = 16
NUM_WORKERS = NUM_CORES * NUM_SUBCORES
B_PER_W = BATCH // NUM_WORKERS
CHUNK = 32
NCHUNKS = B_PER_W // CHUNK
NGBUF = 3
NXBUF = 3


def _body(x_hbm, idx_hbm, tbl_hbm, out_hbm,
          idx_v, xbuf, rbuf, gsem, xsem, ssem):
    wid = lax.axis_index("s") * NUM_CORES + lax.axis_index("c")
    base = wid * B_PER_W
    pltpu.sync_copy(idx_hbm.at[pl.ds(base, B_PER_W)], idx_v)

    def issue_gather(c):
        return pltpu.async_copy(
            tbl_hbm.at[idx_v.at[pl.ds(c * CHUNK, CHUNK)]],
            rbuf.at[c % NGBUF], gsem.at[c % NGBUF])

    def issue_xload(c):
        return pltpu.async_copy(
            x_hbm.at[pl.ds(base + c * CHUNK, CHUNK)],
            xbuf.at[c % NXBUF], xsem.at[c % NXBUF])

    gathers = {}
    xloads = {}
    stores = {}
    for c in range(NGBUF):
        gathers[c] = issue_gather(c)
    for c in range(NXBUF - 1):
        xloads[c] = issue_xload(c)

    for c in range(NCHUNKS):
        bg = c % NGBUF
        bx = c % NXBUF
        gathers.pop(c).wait()
        xloads.pop(c).wait()

        @plsc.parallel_loop(0, CHUNK, step=1)
        def add_row(i):
            for j in range(D_MODEL // LANES):
                sl = pl.ds(j * LANES, LANES)
                plsc.addupdate(xbuf.at[bx, i, sl], rbuf[bg, i, sl])

        stores[c] = pltpu.async_copy(
            xbuf.at[bx], out_hbm.at[pl.ds(base + c * CHUNK, CHUNK)],
            ssem.at[bx])
        if c + NGBUF < NCHUNKS:
            gathers[c + NGBUF] = issue_gather(c + NGBUF)
        if c + NXBUF - 1 < NCHUNKS:
            if c - 1 in stores:
                stores.pop(c - 1).wait()
            xloads[c + NXBUF - 1] = issue_xload(c + NXBUF - 1)
    for c in sorted(stores):
        stores.pop(c).wait()


def kernel(x, context_info, context_emb_weight):
    mesh = plsc.VectorSubcoreMesh(core_axis_name="c", subcore_axis_name="s")
    kfn = functools.partial(
        pl.kernel,
        mesh=mesh,
        out_type=jax.ShapeDtypeStruct((BATCH, D_MODEL), jnp.float32),
        scratch_types=[
            pltpu.VMEM((B_PER_W,), jnp.int32),
            pltpu.VMEM((NXBUF, CHUNK, D_MODEL), jnp.float32),
            pltpu.VMEM((NGBUF, CHUNK, D_MODEL), jnp.float32),
            pltpu.SemaphoreType.DMA((NGBUF,)),
            pltpu.SemaphoreType.DMA((NXBUF,)),
            pltpu.SemaphoreType.DMA((NXBUF,)),
        ],
    )(_body)
    return kfn(x, context_info.astype(jnp.int32), context_emb_weight)
